# Initial kernel scaffold; baseline (speedup 1.0000x reference)
import jax
import jax.numpy as jnp
from jax.experimental import pallas as pl


def kernel(x, edge_index, edge_attr, ptr, nn1_W1, nn1_b1, nn1_W2, nn1_b2, root1, bias1, nn2_W1, nn2_b1, nn2_W2, nn2_b2, root2, bias2, cls_W, cls_b):
    raise NotImplementedError("write your pallas kernel here")


# speedup vs baseline: 2.5071x; 2.5071x over previous
import functools

import numpy as np
import jax
import jax.numpy as jnp
from jax import lax
from jax.experimental import pallas as pl
from jax.experimental.pallas import tpu as pltpu
from jax.experimental.pallas import tpu_sc as plsc

_NC = 2
_NS = 16
_NW = _NC * _NS


def _sc_gather_rows(table, idx):
    n, d = table.shape
    e = idx.shape[0]
    ch = 128
    nchunk = e // ch
    assert nchunk % _NW == 0
    outer = nchunk // _NW
    idx2 = idx.reshape(nchunk, ch)
    mesh = plsc.VectorSubcoreMesh(core_axis_name="c", subcore_axis_name="s")

    @functools.partial(
        pl.kernel,
        out_type=jax.ShapeDtypeStruct((e, d), table.dtype),
        mesh=mesh,
        scratch_types=[
            pltpu.VMEM((ch,), jnp.int32),
            pltpu.VMEM((ch, d), table.dtype),
            pltpu.SemaphoreType.DMA,
        ],
    )
    def k(table_hbm, idx_hbm, out_hbm, idx_v, rows_v, sem):
        w = lax.axis_index("s") * _NC + lax.axis_index("c")

        @pl.loop(0, outer)
        def _(j):
            c = w + j * _NW
            pltpu.sync_copy(idx_hbm.at[c], idx_v)
            pltpu.async_copy(table_hbm.at[idx_v], rows_v, sem).wait()
            pltpu.sync_copy(rows_v, out_hbm.at[pl.ds(c * ch, ch)])

    return k(table, idx2)


def _sc_scatter_edges(msg, dst, n):
    e, d = msg.shape
    ch = 128
    nchunk = e // ch
    assert nchunk % _NW == 0
    outer = nchunk // _NW
    n_pad = (n + 2048) // 2048 * 2048
    rps = n_pad // _NS
    m3 = msg.reshape(nchunk, ch, d)
    d2 = dst.reshape(nchunk, ch)
    mesh = plsc.VectorSubcoreMesh(core_axis_name="c", subcore_axis_name="s")

    @functools.partial(
        pl.kernel,
        out_type=jax.ShapeDtypeStruct((_NC, n_pad, d), jnp.float32),
        mesh=mesh,
        scratch_types=[
            pltpu.VMEM((ch,), jnp.int32),
            pltpu.VMEM((ch, d), jnp.float32),
            pltpu.VMEM_SHARED((n_pad, d), jnp.float32),
        ],
    )
    def k(zero_hbm, msg_hbm, dst_hbm, out_hbm, idx_v, rows_v, acc):
        cid = lax.axis_index("c")
        sid = lax.axis_index("s")
        w = sid * _NC + cid
        r0 = sid * rps
        pltpu.sync_copy(zero_hbm, rows_v)

        @pl.loop(0, rps // ch)
        def _(i):
            pltpu.sync_copy(rows_v, acc.at[pl.ds(r0 + i * ch, ch)])

        plsc.subcore_barrier()

        @pl.loop(0, outer)
        def _(j):
            ci = w + j * _NW
            pltpu.sync_copy(dst_hbm.at[ci], idx_v)
            pltpu.sync_copy(msg_hbm.at[ci], rows_v)
            pltpu.sync_copy(rows_v, acc.at[idx_v], add=True)

        plsc.subcore_barrier()
        pltpu.sync_copy(acc.at[pl.ds(r0, rps)],
                        out_hbm.at[cid, pl.ds(r0, rps)])

    return k(jnp.zeros((ch, d), jnp.float32), m3, d2)


def _tc_edge1_kernel(ea, xj, w1, b1, w2p_bf, b2p, s1):
    e = ea.shape[0]
    in_c = xj.shape[1]
    t = 2048
    grid = e // t

    def body(ea_ref, xj_ref, w1_ref, b1_ref, w2p_ref, b2p_ref, s1_ref,
             msg_ref):
        ea_t = ea_ref[...]
        g1 = jnp.maximum(
            jnp.dot(ea_t, w1_ref[...], preferred_element_type=jnp.float32)
            + b1_ref[...], 0.0)
        h = jnp.dot(g1.astype(jnp.bfloat16), w2p_ref[...],
                    preferred_element_type=jnp.float32) + b2p_ref[...]
        xj_t = xj_ref[...]
        xt = jnp.concatenate([xj_t, xj_t, xj_t, xj_t], axis=1)
        msg = jnp.dot(h * xt, s1_ref[...], preferred_element_type=jnp.float32)
        msg_ref[...] = jnp.concatenate(
            [msg, jnp.ones((t, 1), jnp.float32),
             jnp.zeros((t, 123), jnp.float32)], axis=1)

    full = lambda a: pl.BlockSpec(a.shape, lambda i: (0,) * a.ndim)
    return pl.pallas_call(
        body,
        grid=(grid,),
        in_specs=[
            pl.BlockSpec((t, 4), lambda i: (i, 0)),
            pl.BlockSpec((t, in_c), lambda i: (i, 0)),
            full(w1), full(b1), full(w2p_bf), full(b2p), full(s1),
        ],
        out_specs=pl.BlockSpec((t, 128), lambda i: (i, 0)),
        out_shape=jax.ShapeDtypeStruct((e, 128), jnp.float32),
    )(ea, xj, w1, b1, w2p_bf, b2p, s1)


def _tc_edge2_kernel(ea, xj2, w21, b21, w22p, b22p, s16):
    e = ea.shape[0]
    t = 8192
    grid = e // t

    def body(ea_ref, xj_ref, w21_ref, b21_ref, w22p_ref, b22p_ref, s_ref,
             msg_ref):
        g2 = jnp.maximum(
            jnp.dot(ea_ref[...], w21_ref[...],
                    preferred_element_type=jnp.float32) + b21_ref[...], 0.0)
        w2e = jnp.dot(g2, w22p_ref[...],
                      preferred_element_type=jnp.float32) + b22p_ref[...]
        xj4 = xj_ref[...][:, 0:4]
        xt = jnp.concatenate([xj4, xj4, xj4, xj4], axis=1)
        msg = jnp.dot(w2e * xt, s_ref[...],
                      preferred_element_type=jnp.float32)
        msg_ref[...] = jnp.concatenate(
            [msg, jnp.zeros((t, 124), jnp.float32)], axis=1)

    full = lambda a: pl.BlockSpec(a.shape, lambda i: (0,) * a.ndim)
    return pl.pallas_call(
        body,
        grid=(grid,),
        in_specs=[
            pl.BlockSpec((t, 4), lambda i: (i, 0)),
            pl.BlockSpec((t, 128), lambda i: (i, 0)),
            full(w21), full(b21), full(w22p), full(b22p), full(s16),
        ],
        out_specs=pl.BlockSpec((t, 128), lambda i: (i, 0)),
        out_shape=jax.ShapeDtypeStruct((e, 128), jnp.float32),
    )(ea, xj2, w21, b21, w22p, b22p, s16)


def _tc_layer1_finish(part1, x, root1, bias1):
    n = x.shape[0]

    def body(p_ref, x_ref, r1_ref, b1_ref, h1p_ref):
        p = p_ref[0, 0:n] + p_ref[1, 0:n]
        msum = p[:, 0:4]
        rinv = 1.0 / jnp.maximum(p[:, 4:5], 1.0)
        xr = jnp.dot(x_ref[...], r1_ref[...],
                     preferred_element_type=jnp.float32)
        h1 = jnp.maximum(msum * rinv + xr + b1_ref[...], 0.0)
        h1p_ref[...] = jnp.concatenate(
            [h1, rinv, jnp.zeros((n, 123), jnp.float32)], axis=1)

    return pl.pallas_call(
        body,
        out_shape=jax.ShapeDtypeStruct((n, 128), jnp.float32),
    )(part1, x, root1, bias1)


def _tc_finish(part2, h1p, root2p, bias2, centers, cls_w, cls_b):
    n = h1p.shape[0]
    ng = centers.shape[0]

    def body(p_ref, h1p_ref, r2_ref, b2_ref, c_ref, cw_ref, cb_ref, out_ref):
        p = p_ref[0, 0:n] + p_ref[1, 0:n]
        h1p_t = h1p_ref[...]
        rinv = h1p_t[:, 4:5]
        r2 = jnp.dot(h1p_t, r2_ref[...], preferred_element_type=jnp.float32)
        h2 = jnp.maximum(p[:, 0:4] * rinv + r2 + b2_ref[...], 0.0)
        io = lax.broadcasted_iota(jnp.int32, (ng, n), 1)
        oh = (io == c_ref[...]).astype(jnp.float32)
        cr = jnp.dot(oh, h2, preferred_element_type=jnp.float32)
        out_ref[...] = jnp.dot(cr, cw_ref[...],
                               preferred_element_type=jnp.float32) + cb_ref[...]

    return pl.pallas_call(
        body,
        out_shape=jax.ShapeDtypeStruct((ng, cls_w.shape[1]), jnp.float32),
    )(part2, h1p, root2p, bias2, centers, cls_w, cls_b)


def kernel(x, edge_index, edge_attr, ptr, nn1_W1, nn1_b1, nn1_W2, nn1_b2,
           root1, bias1, nn2_W1, nn2_b1, nn2_W2, nn2_b2, root2, bias2,
           cls_W, cls_b):
    n, in_c = x.shape
    hid = root1.shape[1]
    d1 = in_c * hid
    d2 = hid * hid
    e = edge_index.shape[1]
    e_pad = (e + 8191) // 8192 * 8192
    src = jnp.concatenate(
        [edge_index[0], jnp.zeros((e_pad - e,), jnp.int32)])
    dst = jnp.concatenate(
        [edge_index[1], jnp.full((e_pad - e,), n, jnp.int32)])
    edge_attr = jnp.concatenate(
        [edge_attr, jnp.zeros((e_pad - e, edge_attr.shape[1]),
                              jnp.float32)], axis=0)

    w2p = nn1_W2.reshape(d1, in_c, hid).transpose(0, 2, 1).reshape(d1, d1)
    b2p = nn1_b2.reshape(in_c, hid).T.reshape(1, d1)
    w22p = nn2_W2.reshape(d2, hid, hid).transpose(0, 2, 1).reshape(d2, d2)
    b22p = nn2_b2.reshape(hid, hid).T.reshape(1, d2)
    s1 = jnp.asarray(np.repeat(np.eye(hid, dtype=np.float32), in_c, axis=0))
    s16 = jnp.asarray(np.repeat(np.eye(hid, dtype=np.float32), hid, axis=0))

    xj1 = _sc_gather_rows(x, src)
    msg1 = _tc_edge1_kernel(edge_attr, xj1, nn1_W1, nn1_b1.reshape(1, d1),
                            w2p.astype(jnp.bfloat16), b2p, s1)
    part1 = _sc_scatter_edges(msg1, dst, n)
    h1p = _tc_layer1_finish(part1, x, root1, bias1.reshape(1, hid))

    xj2 = _sc_gather_rows(h1p, src)
    msg2 = _tc_edge2_kernel(edge_attr, xj2, nn2_W1, nn2_b1.reshape(1, d2),
                            w22p, b22p, s16)
    part2 = _sc_scatter_edges(msg2, dst, n)

    root2p = jnp.concatenate(
        [root2, jnp.zeros((128 - hid, hid), jnp.float32)], axis=0)
    return _tc_finish(part2, h1p, root2p, bias2.reshape(1, hid),
                      ptr[:-1].reshape(-1, 1), cls_W, cls_b.reshape(1, -1))


# speedup vs baseline: 2.6438x; 1.0545x over previous
import functools

import numpy as np
import jax
import jax.numpy as jnp
from jax import lax
from jax.experimental import pallas as pl
from jax.experimental.pallas import tpu as pltpu
from jax.experimental.pallas import tpu_sc as plsc

_NC = 2
_NS = 16
_NW = _NC * _NS


def _sc_gather_rows(table, idx):
    n, d = table.shape
    e = idx.shape[0]
    ch = 128
    nchunk = e // ch
    assert nchunk % _NW == 0
    outer = nchunk // _NW
    idx2 = idx.reshape(nchunk, ch)
    mesh = plsc.VectorSubcoreMesh(core_axis_name="c", subcore_axis_name="s")

    @functools.partial(
        pl.kernel,
        out_type=jax.ShapeDtypeStruct((e, d), table.dtype),
        mesh=mesh,
        scratch_types=[
            pltpu.VMEM((ch,), jnp.int32),
            pltpu.VMEM((ch, d), table.dtype),
            pltpu.SemaphoreType.DMA,
        ],
    )
    def k(table_hbm, idx_hbm, out_hbm, idx_v, rows_v, sem):
        w = lax.axis_index("s") * _NC + lax.axis_index("c")

        @pl.loop(0, outer)
        def _(j):
            c = w + j * _NW
            pltpu.sync_copy(idx_hbm.at[c], idx_v)
            pltpu.async_copy(table_hbm.at[idx_v], rows_v, sem).wait()
            pltpu.sync_copy(rows_v, out_hbm.at[pl.ds(c * ch, ch)])

    return k(table, idx2)


def _sc_scatter_edges(msg, dst, n):
    e, d = msg.shape
    ch = 128
    nchunk = e // ch
    assert nchunk % _NW == 0
    outer = nchunk // _NW
    n_pad = (n + 2048) // 2048 * 2048
    rps = n_pad // _NS
    m3 = msg.reshape(nchunk, ch, d)
    d2 = dst.reshape(nchunk, ch)
    mesh = plsc.VectorSubcoreMesh(core_axis_name="c", subcore_axis_name="s")

    assert outer % 2 == 0

    @functools.partial(
        pl.kernel,
        out_type=jax.ShapeDtypeStruct((_NC, n_pad, d), jnp.float32),
        mesh=mesh,
        scratch_types=[
            pltpu.VMEM((ch,), jnp.int32),
            pltpu.VMEM((ch,), jnp.int32),
            pltpu.VMEM((ch, d), jnp.float32),
            pltpu.VMEM((ch, d), jnp.float32),
            pltpu.VMEM_SHARED((n_pad, d), jnp.float32),
            pltpu.SemaphoreType.DMA,
            pltpu.SemaphoreType.DMA,
            pltpu.SemaphoreType.DMA,
            pltpu.SemaphoreType.DMA,
        ],
    )
    def k(zero_hbm, msg_hbm, dst_hbm, out_hbm, idx0, idx1, rows0, rows1,
          acc, si0, si1, sm0, sm1):
        cid = lax.axis_index("c")
        sid = lax.axis_index("s")
        w = sid * _NC + cid
        r0 = sid * rps
        pltpu.sync_copy(zero_hbm, rows0)

        @pl.loop(0, rps // ch)
        def _(i):
            pltpu.sync_copy(rows0, acc.at[pl.ds(r0 + i * ch, ch)])

        plsc.subcore_barrier()

        @pl.loop(0, outer // 2)
        def _(t):
            c0 = w + (2 * t) * _NW
            c1 = w + (2 * t + 1) * _NW
            cpi0 = pltpu.async_copy(dst_hbm.at[c0], idx0, si0)
            cpm0 = pltpu.async_copy(msg_hbm.at[c0], rows0, sm0)
            cpi1 = pltpu.async_copy(dst_hbm.at[c1], idx1, si1)
            cpm1 = pltpu.async_copy(msg_hbm.at[c1], rows1, sm1)
            cpi0.wait()
            cpm0.wait()
            pltpu.sync_copy(rows0, acc.at[idx0], add=True)
            cpi1.wait()
            cpm1.wait()
            pltpu.sync_copy(rows1, acc.at[idx1], add=True)

        plsc.subcore_barrier()
        pltpu.sync_copy(acc.at[pl.ds(r0, rps)],
                        out_hbm.at[cid, pl.ds(r0, rps)])

    return k(jnp.zeros((ch, d), jnp.float32), m3, d2)


def _tc_edge1_kernel(ea, xj, w1, b1, w2p_bf, b2p, s1):
    e = ea.shape[0]
    in_c = xj.shape[1]
    t = 2048
    grid = e // t

    def body(ea_ref, xj_ref, w1_ref, b1_ref, w2p_ref, b2p_ref, s1_ref,
             msg_ref):
        ea_t = ea_ref[...]
        g1 = jnp.maximum(
            jnp.dot(ea_t, w1_ref[...], preferred_element_type=jnp.float32)
            + b1_ref[...], 0.0)
        h = jnp.dot(g1.astype(jnp.bfloat16), w2p_ref[...],
                    preferred_element_type=jnp.float32) + b2p_ref[...]
        xj_t = xj_ref[...]
        xt = jnp.concatenate([xj_t, xj_t, xj_t, xj_t], axis=1)
        msg = jnp.dot(h * xt, s1_ref[...], preferred_element_type=jnp.float32)
        msg_ref[...] = jnp.concatenate(
            [msg, jnp.ones((t, 1), jnp.float32),
             jnp.zeros((t, 123), jnp.float32)], axis=1)

    full = lambda a: pl.BlockSpec(a.shape, lambda i: (0,) * a.ndim)
    return pl.pallas_call(
        body,
        grid=(grid,),
        in_specs=[
            pl.BlockSpec((t, 4), lambda i: (i, 0)),
            pl.BlockSpec((t, in_c), lambda i: (i, 0)),
            full(w1), full(b1), full(w2p_bf), full(b2p), full(s1),
        ],
        out_specs=pl.BlockSpec((t, 128), lambda i: (i, 0)),
        out_shape=jax.ShapeDtypeStruct((e, 128), jnp.float32),
    )(ea, xj, w1, b1, w2p_bf, b2p, s1)


def _tc_edge2_kernel(ea, xj2, w21, b21, w22p, b22p, s16):
    e = ea.shape[0]
    t = 8192
    grid = e // t

    def body(ea_ref, xj_ref, w21_ref, b21_ref, w22p_ref, b22p_ref, s_ref,
             msg_ref):
        g2 = jnp.maximum(
            jnp.dot(ea_ref[...], w21_ref[...],
                    preferred_element_type=jnp.float32) + b21_ref[...], 0.0)
        w2e = jnp.dot(g2, w22p_ref[...],
                      preferred_element_type=jnp.float32) + b22p_ref[...]
        xj4 = xj_ref[...][:, 0:4]
        xt = jnp.concatenate([xj4, xj4, xj4, xj4], axis=1)
        msg = jnp.dot(w2e * xt, s_ref[...],
                      preferred_element_type=jnp.float32)
        msg_ref[...] = jnp.concatenate(
            [msg, jnp.zeros((t, 124), jnp.float32)], axis=1)

    full = lambda a: pl.BlockSpec(a.shape, lambda i: (0,) * a.ndim)
    return pl.pallas_call(
        body,
        grid=(grid,),
        in_specs=[
            pl.BlockSpec((t, 4), lambda i: (i, 0)),
            pl.BlockSpec((t, 128), lambda i: (i, 0)),
            full(w21), full(b21), full(w22p), full(b22p), full(s16),
        ],
        out_specs=pl.BlockSpec((t, 128), lambda i: (i, 0)),
        out_shape=jax.ShapeDtypeStruct((e, 128), jnp.float32),
    )(ea, xj2, w21, b21, w22p, b22p, s16)


def _tc_layer1_finish(part1, x, root1, bias1):
    n = x.shape[0]

    def body(p_ref, x_ref, r1_ref, b1_ref, h1p_ref):
        p = p_ref[0, 0:n] + p_ref[1, 0:n]
        msum = p[:, 0:4]
        rinv = 1.0 / jnp.maximum(p[:, 4:5], 1.0)
        xr = jnp.dot(x_ref[...], r1_ref[...],
                     preferred_element_type=jnp.float32)
        h1 = jnp.maximum(msum * rinv + xr + b1_ref[...], 0.0)
        h1p_ref[...] = jnp.concatenate(
            [h1, rinv, jnp.zeros((n, 123), jnp.float32)], axis=1)

    return pl.pallas_call(
        body,
        out_shape=jax.ShapeDtypeStruct((n, 128), jnp.float32),
    )(part1, x, root1, bias1)


def _tc_finish(part2, h1p, root2p, bias2, centers, cls_w, cls_b):
    n = h1p.shape[0]
    ng = centers.shape[0]

    def body(p_ref, h1p_ref, r2_ref, b2_ref, c_ref, cw_ref, cb_ref, out_ref):
        p = p_ref[0, 0:n] + p_ref[1, 0:n]
        h1p_t = h1p_ref[...]
        rinv = h1p_t[:, 4:5]
        r2 = jnp.dot(h1p_t, r2_ref[...], preferred_element_type=jnp.float32)
        h2 = jnp.maximum(p[:, 0:4] * rinv + r2 + b2_ref[...], 0.0)
        io = lax.broadcasted_iota(jnp.int32, (ng, n), 1)
        oh = (io == c_ref[...]).astype(jnp.float32)
        cr = jnp.dot(oh, h2, preferred_element_type=jnp.float32)
        out_ref[...] = jnp.dot(cr, cw_ref[...],
                               preferred_element_type=jnp.float32) + cb_ref[...]

    return pl.pallas_call(
        body,
        out_shape=jax.ShapeDtypeStruct((ng, cls_w.shape[1]), jnp.float32),
    )(part2, h1p, root2p, bias2, centers, cls_w, cls_b)


def kernel(x, edge_index, edge_attr, ptr, nn1_W1, nn1_b1, nn1_W2, nn1_b2,
           root1, bias1, nn2_W1, nn2_b1, nn2_W2, nn2_b2, root2, bias2,
           cls_W, cls_b):
    n, in_c = x.shape
    hid = root1.shape[1]
    d1 = in_c * hid
    d2 = hid * hid
    e = edge_index.shape[1]
    e_pad = (e + 8191) // 8192 * 8192
    src = jnp.concatenate(
        [edge_index[0], jnp.zeros((e_pad - e,), jnp.int32)])
    dst = jnp.concatenate(
        [edge_index[1], jnp.full((e_pad - e,), n, jnp.int32)])
    edge_attr = jnp.concatenate(
        [edge_attr, jnp.zeros((e_pad - e, edge_attr.shape[1]),
                              jnp.float32)], axis=0)

    w2p = nn1_W2.reshape(d1, in_c, hid).transpose(0, 2, 1).reshape(d1, d1)
    b2p = nn1_b2.reshape(in_c, hid).T.reshape(1, d1)
    w22p = nn2_W2.reshape(d2, hid, hid).transpose(0, 2, 1).reshape(d2, d2)
    b22p = nn2_b2.reshape(hid, hid).T.reshape(1, d2)
    s1 = jnp.asarray(np.repeat(np.eye(hid, dtype=np.float32), in_c, axis=0))
    s16 = jnp.asarray(np.repeat(np.eye(hid, dtype=np.float32), hid, axis=0))

    xj1 = _sc_gather_rows(x, src)
    msg1 = _tc_edge1_kernel(edge_attr, xj1, nn1_W1, nn1_b1.reshape(1, d1),
                            w2p.astype(jnp.bfloat16), b2p, s1)
    part1 = _sc_scatter_edges(msg1, dst, n)
    h1p = _tc_layer1_finish(part1, x, root1, bias1.reshape(1, hid))

    xj2 = _sc_gather_rows(h1p, src)
    msg2 = _tc_edge2_kernel(edge_attr, xj2, nn2_W1, nn2_b1.reshape(1, d2),
                            w22p, b22p, s16)
    part2 = _sc_scatter_edges(msg2, dst, n)

    root2p = jnp.concatenate(
        [root2, jnp.zeros((128 - hid, hid), jnp.float32)], axis=0)
    return _tc_finish(part2, h1p, root2p, bias2.reshape(1, hid),
                      ptr[:-1].reshape(-1, 1), cls_W, cls_b.reshape(1, -1))


# speedup vs baseline: 2.7069x; 1.0239x over previous
import functools

import numpy as np
import jax
import jax.numpy as jnp
from jax import lax
from jax.experimental import pallas as pl
from jax.experimental.pallas import tpu as pltpu
from jax.experimental.pallas import tpu_sc as plsc

_NC = 2
_NS = 16
_NW = _NC * _NS


def _sc_gather_rows(table, idx):
    n, d = table.shape
    e = idx.shape[0]
    ch = 128
    nchunk = e // ch
    assert nchunk % _NW == 0
    outer = nchunk // _NW
    idx2 = idx.reshape(nchunk, ch)
    mesh = plsc.VectorSubcoreMesh(core_axis_name="c", subcore_axis_name="s")

    assert outer % 2 == 0

    @functools.partial(
        pl.kernel,
        out_type=jax.ShapeDtypeStruct((e, d), table.dtype),
        mesh=mesh,
        scratch_types=[
            pltpu.VMEM((ch,), jnp.int32),
            pltpu.VMEM((ch,), jnp.int32),
            pltpu.VMEM((ch, d), table.dtype),
            pltpu.VMEM((ch, d), table.dtype),
            pltpu.SemaphoreType.DMA,
            pltpu.SemaphoreType.DMA,
            pltpu.SemaphoreType.DMA,
            pltpu.SemaphoreType.DMA,
        ],
    )
    def k(table_hbm, idx_hbm, out_hbm, idx0, idx1, rows0, rows1,
          si0, si1, sg0, sg1):
        w = lax.axis_index("s") * _NC + lax.axis_index("c")

        @pl.loop(0, outer // 2)
        def _(t):
            c0 = w + (2 * t) * _NW
            c1 = w + (2 * t + 1) * _NW
            cpi0 = pltpu.async_copy(idx_hbm.at[c0], idx0, si0)
            cpi1 = pltpu.async_copy(idx_hbm.at[c1], idx1, si1)
            cpi0.wait()
            g0 = pltpu.async_copy(table_hbm.at[idx0], rows0, sg0)
            cpi1.wait()
            g1 = pltpu.async_copy(table_hbm.at[idx1], rows1, sg1)
            g0.wait()
            pltpu.sync_copy(rows0, out_hbm.at[pl.ds(c0 * ch, ch)])
            g1.wait()
            pltpu.sync_copy(rows1, out_hbm.at[pl.ds(c1 * ch, ch)])

    return k(table, idx2)


def _sc_scatter_edges(msg, dst, n):
    e, d = msg.shape
    ch = 128
    nchunk = e // ch
    assert nchunk % _NW == 0
    outer = nchunk // _NW
    n_pad = (n + 2048) // 2048 * 2048
    rps = n_pad // _NS
    m3 = msg.reshape(nchunk, ch, d)
    d2 = dst.reshape(nchunk, ch)
    mesh = plsc.VectorSubcoreMesh(core_axis_name="c", subcore_axis_name="s")

    assert outer % 2 == 0

    @functools.partial(
        pl.kernel,
        out_type=jax.ShapeDtypeStruct((_NC, n_pad, d), jnp.float32),
        mesh=mesh,
        scratch_types=[
            pltpu.VMEM((ch,), jnp.int32),
            pltpu.VMEM((ch,), jnp.int32),
            pltpu.VMEM((ch, d), jnp.float32),
            pltpu.VMEM((ch, d), jnp.float32),
            pltpu.VMEM_SHARED((n_pad, d), jnp.float32),
            pltpu.SemaphoreType.DMA,
            pltpu.SemaphoreType.DMA,
            pltpu.SemaphoreType.DMA,
            pltpu.SemaphoreType.DMA,
        ],
    )
    def k(zero_hbm, msg_hbm, dst_hbm, out_hbm, idx0, idx1, rows0, rows1,
          acc, si0, si1, sm0, sm1):
        cid = lax.axis_index("c")
        sid = lax.axis_index("s")
        w = sid * _NC + cid
        r0 = sid * rps
        pltpu.sync_copy(zero_hbm, rows0)

        @pl.loop(0, rps // ch)
        def _(i):
            pltpu.sync_copy(rows0, acc.at[pl.ds(r0 + i * ch, ch)])

        plsc.subcore_barrier()

        @pl.loop(0, outer // 2)
        def _(t):
            c0 = w + (2 * t) * _NW
            c1 = w + (2 * t + 1) * _NW
            cpi0 = pltpu.async_copy(dst_hbm.at[c0], idx0, si0)
            cpm0 = pltpu.async_copy(msg_hbm.at[c0], rows0, sm0)
            cpi1 = pltpu.async_copy(dst_hbm.at[c1], idx1, si1)
            cpm1 = pltpu.async_copy(msg_hbm.at[c1], rows1, sm1)
            cpi0.wait()
            cpm0.wait()
            pltpu.sync_copy(rows0, acc.at[idx0], add=True)
            cpi1.wait()
            cpm1.wait()
            pltpu.sync_copy(rows1, acc.at[idx1], add=True)

        plsc.subcore_barrier()
        pltpu.sync_copy(acc.at[pl.ds(r0, rps)],
                        out_hbm.at[cid, pl.ds(r0, rps)])

    return k(jnp.zeros((ch, d), jnp.float32), m3, d2)


def _tc_edge1_kernel(ea, xj, w1, b1, w2p_bf, b2p, s1):
    e = ea.shape[0]
    in_c = xj.shape[1]
    t = 2048
    grid = e // t

    def body(ea_ref, xj_ref, w1_ref, b1_ref, w2p_ref, b2p_ref, s1_ref,
             msg_ref):
        ea_t = ea_ref[...]
        g1 = jnp.maximum(
            jnp.dot(ea_t, w1_ref[...], preferred_element_type=jnp.float32)
            + b1_ref[...], 0.0)
        h = jnp.dot(g1.astype(jnp.bfloat16), w2p_ref[...],
                    preferred_element_type=jnp.float32) + b2p_ref[...]
        xj_t = xj_ref[...]
        xt = jnp.concatenate([xj_t, xj_t, xj_t, xj_t], axis=1)
        msg = jnp.dot(h * xt, s1_ref[...], preferred_element_type=jnp.float32)
        msg_ref[...] = jnp.concatenate(
            [msg, jnp.ones((t, 1), jnp.float32),
             jnp.zeros((t, 123), jnp.float32)], axis=1)

    full = lambda a: pl.BlockSpec(a.shape, lambda i: (0,) * a.ndim)
    return pl.pallas_call(
        body,
        grid=(grid,),
        in_specs=[
            pl.BlockSpec((t, 4), lambda i: (i, 0)),
            pl.BlockSpec((t, in_c), lambda i: (i, 0)),
            full(w1), full(b1), full(w2p_bf), full(b2p), full(s1),
        ],
        out_specs=pl.BlockSpec((t, 128), lambda i: (i, 0)),
        out_shape=jax.ShapeDtypeStruct((e, 128), jnp.float32),
    )(ea, xj, w1, b1, w2p_bf, b2p, s1)


def _tc_edge2_kernel(ea, xj2, w21, b21, w22p, b22p, s16):
    e = ea.shape[0]
    t = 8192
    grid = e // t

    def body(ea_ref, xj_ref, w21_ref, b21_ref, w22p_ref, b22p_ref, s_ref,
             msg_ref):
        g2 = jnp.maximum(
            jnp.dot(ea_ref[...], w21_ref[...],
                    preferred_element_type=jnp.float32) + b21_ref[...], 0.0)
        w2e = jnp.dot(g2, w22p_ref[...],
                      preferred_element_type=jnp.float32) + b22p_ref[...]
        xj4 = xj_ref[...][:, 0:4]
        xt = jnp.concatenate([xj4, xj4, xj4, xj4], axis=1)
        msg = jnp.dot(w2e * xt, s_ref[...],
                      preferred_element_type=jnp.float32)
        msg_ref[...] = jnp.concatenate(
            [msg, jnp.zeros((t, 124), jnp.float32)], axis=1)

    full = lambda a: pl.BlockSpec(a.shape, lambda i: (0,) * a.ndim)
    return pl.pallas_call(
        body,
        grid=(grid,),
        in_specs=[
            pl.BlockSpec((t, 4), lambda i: (i, 0)),
            pl.BlockSpec((t, 128), lambda i: (i, 0)),
            full(w21), full(b21), full(w22p), full(b22p), full(s16),
        ],
        out_specs=pl.BlockSpec((t, 128), lambda i: (i, 0)),
        out_shape=jax.ShapeDtypeStruct((e, 128), jnp.float32),
    )(ea, xj2, w21, b21, w22p, b22p, s16)


def _tc_layer1_finish(part1, x, root1, bias1):
    n = x.shape[0]

    def body(p_ref, x_ref, r1_ref, b1_ref, h1p_ref):
        p = p_ref[0, 0:n] + p_ref[1, 0:n]
        msum = p[:, 0:4]
        rinv = 1.0 / jnp.maximum(p[:, 4:5], 1.0)
        xr = jnp.dot(x_ref[...], r1_ref[...],
                     preferred_element_type=jnp.float32)
        h1 = jnp.maximum(msum * rinv + xr + b1_ref[...], 0.0)
        h1p_ref[...] = jnp.concatenate(
            [h1, rinv, jnp.zeros((n, 123), jnp.float32)], axis=1)

    return pl.pallas_call(
        body,
        out_shape=jax.ShapeDtypeStruct((n, 128), jnp.float32),
    )(part1, x, root1, bias1)


def _tc_finish(part2, h1p, root2p, bias2, centers, cls_w, cls_b):
    n = h1p.shape[0]
    ng = centers.shape[0]

    def body(p_ref, h1p_ref, r2_ref, b2_ref, c_ref, cw_ref, cb_ref, out_ref):
        p = p_ref[0, 0:n] + p_ref[1, 0:n]
        h1p_t = h1p_ref[...]
        rinv = h1p_t[:, 4:5]
        r2 = jnp.dot(h1p_t, r2_ref[...], preferred_element_type=jnp.float32)
        h2 = jnp.maximum(p[:, 0:4] * rinv + r2 + b2_ref[...], 0.0)
        io = lax.broadcasted_iota(jnp.int32, (ng, n), 1)
        oh = (io == c_ref[...]).astype(jnp.float32)
        cr = jnp.dot(oh, h2, preferred_element_type=jnp.float32)
        out_ref[...] = jnp.dot(cr, cw_ref[...],
                               preferred_element_type=jnp.float32) + cb_ref[...]

    return pl.pallas_call(
        body,
        out_shape=jax.ShapeDtypeStruct((ng, cls_w.shape[1]), jnp.float32),
    )(part2, h1p, root2p, bias2, centers, cls_w, cls_b)


def kernel(x, edge_index, edge_attr, ptr, nn1_W1, nn1_b1, nn1_W2, nn1_b2,
           root1, bias1, nn2_W1, nn2_b1, nn2_W2, nn2_b2, root2, bias2,
           cls_W, cls_b):
    n, in_c = x.shape
    hid = root1.shape[1]
    d1 = in_c * hid
    d2 = hid * hid
    e = edge_index.shape[1]
    e_pad = (e + 8191) // 8192 * 8192
    src = jnp.concatenate(
        [edge_index[0], jnp.zeros((e_pad - e,), jnp.int32)])
    dst = jnp.concatenate(
        [edge_index[1], jnp.full((e_pad - e,), n, jnp.int32)])
    edge_attr = jnp.concatenate(
        [edge_attr, jnp.zeros((e_pad - e, edge_attr.shape[1]),
                              jnp.float32)], axis=0)

    w2p = nn1_W2.reshape(d1, in_c, hid).transpose(0, 2, 1).reshape(d1, d1)
    b2p = nn1_b2.reshape(in_c, hid).T.reshape(1, d1)
    w22p = nn2_W2.reshape(d2, hid, hid).transpose(0, 2, 1).reshape(d2, d2)
    b22p = nn2_b2.reshape(hid, hid).T.reshape(1, d2)
    s1 = jnp.asarray(np.repeat(np.eye(hid, dtype=np.float32), in_c, axis=0))
    s16 = jnp.asarray(np.repeat(np.eye(hid, dtype=np.float32), hid, axis=0))

    xj1 = _sc_gather_rows(x, src)
    msg1 = _tc_edge1_kernel(edge_attr, xj1, nn1_W1, nn1_b1.reshape(1, d1),
                            w2p.astype(jnp.bfloat16), b2p, s1)
    part1 = _sc_scatter_edges(msg1, dst, n)
    h1p = _tc_layer1_finish(part1, x, root1, bias1.reshape(1, hid))

    xj2 = _sc_gather_rows(h1p, src)
    msg2 = _tc_edge2_kernel(edge_attr, xj2, nn2_W1, nn2_b1.reshape(1, d2),
                            w22p, b22p, s16)
    part2 = _sc_scatter_edges(msg2, dst, n)

    root2p = jnp.concatenate(
        [root2, jnp.zeros((128 - hid, hid), jnp.float32)], axis=0)
    return _tc_finish(part2, h1p, root2p, bias2.reshape(1, hid),
                      ptr[:-1].reshape(-1, 1), cls_W, cls_b.reshape(1, -1))
